# Initial kernel scaffold; baseline (speedup 1.0000x reference)
import jax
import jax.numpy as jnp
from jax.experimental import pallas as pl


def kernel(x_source, x_user, edge_index_su, edge_index_us, batch_source, batch_user, W_proj_source, b_proj_source, W_proj_user, b_proj_user, att_src_su, att_dst_su, att_src_us, att_dst_us, q, Wk, bk, W_lin, b_lin, W_cls, b_cls):
    raise NotImplementedError("write your pallas kernel here")


# speedup vs baseline: 12.6430x; 12.6430x over previous
import functools

import jax
import jax.numpy as jnp
from jax import lax
from jax.experimental import pallas as pl
from jax.experimental.pallas import tpu as pltpu
from jax.experimental.pallas import tpu_sc as plsc

N = 10000
C = 128
H = 8
D = 16
E = 320000
G = 64

ROW = 144
NB = 10
BN = N // NB
NC = 2
NS = 16
NW = NC * NS
PER_W = E // NW
CH = 80
NCHUNK = PER_W // CH
NP = 10240
RPT = NP // NS

_f32 = jnp.float32


def _proj_body(xs_ref, xu_ref, ws_ref, wu_ref, bs_ref, bu_ref,
               sbig_ref, ssml_ref, ubig_ref, usml_ref):
    rs = jnp.dot(xs_ref[...], ws_ref[...], preferred_element_type=_f32,
                 precision=lax.Precision.HIGHEST) + bs_ref[...]
    ru = jnp.dot(xu_ref[...], wu_ref[...], preferred_element_type=_f32,
                 precision=lax.Precision.HIGHEST) + bu_ref[...]
    sbig_ref[...] = rs[:, :ROW]
    ssml_ref[...] = rs[:, ROW:ROW + 16]
    ubig_ref[...] = ru[:, :ROW]
    usml_ref[...] = ru[:, ROW:ROW + 16]


def _project(x_s, x_u, wb_s, wb_u, bb_s, bb_u):
    return pl.pallas_call(
        _proj_body,
        grid=(NB,),
        in_specs=[
            pl.BlockSpec((BN, C), lambda i: (i, 0)),
            pl.BlockSpec((BN, C), lambda i: (i, 0)),
            pl.BlockSpec((C, ROW + 16), lambda i: (0, 0)),
            pl.BlockSpec((C, ROW + 16), lambda i: (0, 0)),
            pl.BlockSpec((1, ROW + 16), lambda i: (0, 0)),
            pl.BlockSpec((1, ROW + 16), lambda i: (0, 0)),
        ],
        out_specs=[
            pl.BlockSpec((BN, ROW), lambda i: (i, 0)),
            pl.BlockSpec((BN, 16), lambda i: (i, 0)),
            pl.BlockSpec((BN, ROW), lambda i: (i, 0)),
            pl.BlockSpec((BN, 16), lambda i: (i, 0)),
        ],
        out_shape=[
            jax.ShapeDtypeStruct((N, ROW), _f32),
            jax.ShapeDtypeStruct((N, 16), _f32),
            jax.ShapeDtypeStruct((N, ROW), _f32),
            jax.ShapeDtypeStruct((N, 16), _f32),
        ],
    )(x_s, x_u, wb_s, wb_u, bb_s, bb_u)


def _splat(v, h):
    idx = jnp.full((16, 1), h, jnp.int32)
    return lax.gather(
        v, idx,
        lax.GatherDimensionNumbers(offset_dims=(), collapsed_slice_dims=(0,),
                                   start_index_map=(0,)),
        (1,), mode=lax.GatherScatterMode.PROMISE_IN_BOUNDS)


def _sc_edge_body(big_hbm, small_hbm, sidx_hbm, didx_hbm, zeros_hbm, out_hbm,
                  sidx_v, didx_v, big_v, small_v, sem, acc):
    cid = lax.axis_index("c")
    sid = lax.axis_index("s")
    wid = sid * NC + cid

    pltpu.sync_copy(zeros_hbm, acc.at[pl.ds(sid * RPT, RPT)])
    plsc.subcore_barrier()

    base_w = wid * PER_W

    def chunk_body(ci, carry):
        base = base_w + ci * CH
        pltpu.sync_copy(sidx_hbm.at[pl.ds(base, CH)], sidx_v)
        pltpu.sync_copy(didx_hbm.at[pl.ds(base, CH)], didx_v)
        cp1 = pltpu.async_copy(big_hbm.at[sidx_v], big_v, sem)
        cp2 = pltpu.async_copy(small_hbm.at[didx_v], small_v, sem)
        cp1.wait()
        cp2.wait()

        def edge_body(e, c2):
            t = big_v[e, pl.ds(C, 16)] + small_v[e, pl.ds(0, 16)]
            w = jnp.exp(jnp.maximum(t, 0.2 * t))
            for h in range(H):
                sp = _splat(w, h)
                big_v[e, pl.ds(D * h, D)] = big_v[e, pl.ds(D * h, D)] * sp
            big_v[e, pl.ds(C, 16)] = w
            return c2

        lax.fori_loop(0, CH, edge_body, 0)
        pltpu.sync_copy(big_v, acc.at[didx_v], add=True)
        return carry

    lax.fori_loop(0, NCHUNK, chunk_body, 0)

    plsc.subcore_barrier()
    pltpu.sync_copy(acc.at[pl.ds(sid * RPT, RPT)],
                    out_hbm.at[cid, pl.ds(sid * RPT, RPT)])


_sc_edge = pl.kernel(
    _sc_edge_body,
    out_type=jax.ShapeDtypeStruct((NC, NP, ROW), _f32),
    mesh=plsc.VectorSubcoreMesh(core_axis_name="c", subcore_axis_name="s"),
    scratch_types=[
        pltpu.VMEM((CH,), jnp.int32),
        pltpu.VMEM((CH,), jnp.int32),
        pltpu.VMEM((CH, ROW), _f32),
        pltpu.VMEM((CH, 16), _f32),
        pltpu.SemaphoreType.DMA,
        pltpu.VMEM_SHARED((NP, ROW), _f32),
    ],
    compiler_params=pltpu.CompilerParams(use_tc_tiling_on_sc=False),
)


def _finish_body(pu_ref, ps_ref, bu_ref, bs_ref, wl_ref, bl_ref, wc_ref,
                 bc_ref, out_ref, accs, accu, cnts, cntu):
    i = pl.program_id(0)

    @pl.when(i == 0)
    def _init():
        accs[...] = jnp.zeros_like(accs)
        accu[...] = jnp.zeros_like(accu)
        cnts[...] = jnp.zeros_like(cnts)
        cntu[...] = jnp.zeros_like(cntu)

    rows = lax.broadcasted_iota(jnp.int32, (H, C), 0)
    cols = lax.broadcasted_iota(jnp.int32, (H, C), 1)
    expand = (cols // D == rows).astype(_f32)

    def emb_of(p_ref):
        part = p_ref[0] + p_ref[1]
        num = part[:, :C]
        den = jnp.dot(part[:, C:C + H], expand, preferred_element_type=_f32,
                      precision=lax.Precision.HIGHEST)
        return jnp.maximum(num / (den + 1e-16), 0.0)

    emb_u = emb_of(pu_ref)
    emb_s = emb_of(ps_ref)

    giota = lax.broadcasted_iota(jnp.int32, (G, BN), 0)
    mask_u = (bu_ref[0] == giota).astype(_f32)
    mask_s = (bs_ref[0] == giota).astype(_f32)

    accu[...] += jnp.dot(mask_u, emb_u, preferred_element_type=_f32,
                         precision=lax.Precision.HIGHEST)
    accs[...] += jnp.dot(mask_s, emb_s, preferred_element_type=_f32,
                         precision=lax.Precision.HIGHEST)
    cntu[...] += jnp.broadcast_to(jnp.sum(mask_u, axis=1, keepdims=True),
                                  (G, C))
    cnts[...] += jnp.broadcast_to(jnp.sum(mask_s, axis=1, keepdims=True),
                                  (G, C))

    @pl.when(i == NB - 1)
    def _final():
        ps = accs[...] / jnp.maximum(cnts[...], 1.0)
        pu = accu[...] / jnp.maximum(cntu[...], 1.0)
        pooled = (ps + pu) * 0.5
        x = jnp.dot(pooled, wl_ref[...], preferred_element_type=_f32,
                    precision=lax.Precision.HIGHEST) + bl_ref[...]
        x = jnp.where(x > 0, x, 0.01 * x)
        out_ref[...] = jnp.dot(x, wc_ref[...], preferred_element_type=_f32,
                               precision=lax.Precision.HIGHEST) + bc_ref[...]


def _finish(p_user, p_source, batch_u3, batch_s3, w_lin, b_lin, w_cls, b_cls):
    return pl.pallas_call(
        _finish_body,
        grid=(NB,),
        in_specs=[
            pl.BlockSpec((NC, BN, ROW), lambda i: (0, i, 0)),
            pl.BlockSpec((NC, BN, ROW), lambda i: (0, i, 0)),
            pl.BlockSpec((1, 1, BN), lambda i: (i, 0, 0)),
            pl.BlockSpec((1, 1, BN), lambda i: (i, 0, 0)),
            pl.BlockSpec((C, C), lambda i: (0, 0)),
            pl.BlockSpec((1, C), lambda i: (0, 0)),
            pl.BlockSpec((C, 2), lambda i: (0, 0)),
            pl.BlockSpec((1, 2), lambda i: (0, 0)),
        ],
        out_specs=pl.BlockSpec((G, 2), lambda i: (0, 0)),
        out_shape=jax.ShapeDtypeStruct((G, 2), _f32),
        scratch_shapes=[pltpu.VMEM((G, C), _f32)] * 4,
    )(p_user, p_source, batch_u3, batch_s3, w_lin, b_lin, w_cls, b_cls)


def _att_mat(att):
    return (jnp.eye(H, dtype=_f32)[:, None, :] * att[:, :, None]).reshape(C, H)


def kernel(x_source, x_user, edge_index_su, edge_index_us, batch_source,
           batch_user, W_proj_source, b_proj_source, W_proj_user, b_proj_user,
           att_src_su, att_dst_su, att_src_us, att_dst_us, q, Wk, bk, W_lin,
           b_lin, W_cls, b_cls):
    z8 = jnp.zeros((C, H), _f32)
    wb_s = jnp.concatenate(
        [W_proj_source, W_proj_source @ _att_mat(att_src_su), z8,
         W_proj_source @ _att_mat(att_dst_us), z8], axis=1)
    wb_u = jnp.concatenate(
        [W_proj_user, W_proj_user @ _att_mat(att_src_us), z8,
         W_proj_user @ _att_mat(att_dst_su), z8], axis=1)
    zb = jnp.zeros((H,), _f32)
    bb_s = jnp.concatenate([b_proj_source, b_proj_source @ _att_mat(att_src_su),
                            zb, b_proj_source @ _att_mat(att_dst_us), zb])
    bb_u = jnp.concatenate([b_proj_user, b_proj_user @ _att_mat(att_src_us),
                            zb, b_proj_user @ _att_mat(att_dst_su), zb])

    s_big, s_small, u_big, u_small = _project(
        x_source, x_user, wb_s, wb_u, bb_s[None], bb_u[None])

    zeros_tile = jnp.zeros((RPT, ROW), _f32)
    p_user = _sc_edge(s_big, u_small, edge_index_su[0], edge_index_su[1],
                      zeros_tile)
    p_source = _sc_edge(u_big, s_small, edge_index_us[0], edge_index_us[1],
                        zeros_tile)

    return _finish(p_user, p_source,
                   batch_user.reshape(NB, 1, BN),
                   batch_source.reshape(NB, 1, BN),
                   W_lin, b_lin[None], W_cls, b_cls[None])


# speedup vs baseline: 26.0760x; 2.0625x over previous
import functools

import jax
import jax.numpy as jnp
from jax import lax
from jax.experimental import pallas as pl
from jax.experimental.pallas import tpu as pltpu
from jax.experimental.pallas import tpu_sc as plsc

N = 10000
C = 128
H = 8
D = 16
E = 320000
G = 64

ROW = 144
NB = 10
BN = N // NB
NC = 2
NS = 16
NW = NC * NS
PER_W = E // NW
CH = 40
NCHUNK = PER_W // CH
NPAIR = NCHUNK // 2
NP = 10240
RPT = NP // NS

_f32 = jnp.float32


def _proj_body(xs_ref, xu_ref, ws_ref, wu_ref, bs_ref, bu_ref,
               sbig_ref, ssml_ref, ubig_ref, usml_ref):
    rs = jnp.dot(xs_ref[...], ws_ref[...], preferred_element_type=_f32,
                 precision=lax.Precision.HIGHEST) + bs_ref[...]
    ru = jnp.dot(xu_ref[...], wu_ref[...], preferred_element_type=_f32,
                 precision=lax.Precision.HIGHEST) + bu_ref[...]
    sbig_ref[...] = rs[:, :ROW]
    ssml_ref[...] = rs[:, ROW:ROW + 16]
    ubig_ref[...] = ru[:, :ROW]
    usml_ref[...] = ru[:, ROW:ROW + 16]


def _project(x_s, x_u, wb_s, wb_u, bb_s, bb_u):
    return pl.pallas_call(
        _proj_body,
        grid=(NB,),
        in_specs=[
            pl.BlockSpec((BN, C), lambda i: (i, 0)),
            pl.BlockSpec((BN, C), lambda i: (i, 0)),
            pl.BlockSpec((C, ROW + 16), lambda i: (0, 0)),
            pl.BlockSpec((C, ROW + 16), lambda i: (0, 0)),
            pl.BlockSpec((1, ROW + 16), lambda i: (0, 0)),
            pl.BlockSpec((1, ROW + 16), lambda i: (0, 0)),
        ],
        out_specs=[
            pl.BlockSpec((BN, ROW), lambda i: (i, 0)),
            pl.BlockSpec((BN, 16), lambda i: (i, 0)),
            pl.BlockSpec((BN, ROW), lambda i: (i, 0)),
            pl.BlockSpec((BN, 16), lambda i: (i, 0)),
        ],
        out_shape=[
            jax.ShapeDtypeStruct((N, ROW), _f32),
            jax.ShapeDtypeStruct((N, 16), _f32),
            jax.ShapeDtypeStruct((N, ROW), _f32),
            jax.ShapeDtypeStruct((N, 16), _f32),
        ],
    )(x_s, x_u, wb_s, wb_u, bb_s, bb_u)


def _splat(v, h):
    idx = jnp.full((16, 1), h, jnp.int32)
    return lax.gather(
        v, idx,
        lax.GatherDimensionNumbers(offset_dims=(), collapsed_slice_dims=(0,),
                                   start_index_map=(0,)),
        (1,), mode=lax.GatherScatterMode.PROMISE_IN_BOUNDS)


def _sc_edge_body(big_hbm, small_hbm, sidx_hbm, didx_hbm, zeros_hbm, out_hbm,
                  sidx_all, didx_all, big0, small0, big1, small1,
                  semg0, semg1, semc0, semc1, acc):
    cid = lax.axis_index("c")
    sid = lax.axis_index("s")
    wid = sid * NC + cid

    pltpu.sync_copy(zeros_hbm, acc.at[pl.ds(sid * RPT, RPT)])
    pltpu.sync_copy(sidx_hbm.at[wid], sidx_all)
    pltpu.sync_copy(didx_hbm.at[wid], didx_all)
    plsc.subcore_barrier()

    bufs = ((big0, small0, semg0, semc0), (big1, small1, semg1, semc1))

    def gathers(ci, b):
        big_v, small_v, semg, _ = bufs[b]
        return (pltpu.make_async_copy(big_hbm.at[sidx_all.at[ci]], big_v, semg),
                pltpu.make_async_copy(small_hbm.at[didx_all.at[ci]], small_v,
                                      semg))

    def scatter(ci, b):
        big_v, _, _, semc = bufs[b]
        return pltpu.make_async_copy(big_v, acc.at[didx_all.at[ci]], semc)

    def issue_gathers(ci, b):
        g1, g2 = gathers(ci, b)
        g1.start()
        g2.start()

    def compute(b):
        big_v, small_v, _, _ = bufs[b]

        @plsc.parallel_loop(0, CH, unroll=4)
        def _edges(e):
            t = big_v[e, pl.ds(C, 16)] + small_v[e, pl.ds(0, 16)]
            w = jnp.exp(jnp.maximum(t, 0.2 * t))
            for h in range(H):
                sp = _splat(w, h)
                big_v[e, pl.ds(D * h, D)] = big_v[e, pl.ds(D * h, D)] * sp
            big_v[e, pl.ds(C, 16)] = w

    def pair(j, carry):
        c0 = 2 * j
        for b in range(2):
            ci = c0 + b
            g1, g2 = gathers(ci, b)
            g1.wait()
            g2.wait()
            compute(b)
            scatter(ci, b).start(add=True)

        @pl.when(j < NPAIR - 1)
        def _prefetch():
            for b in range(2):
                scatter(c0 + b, b).wait()
                issue_gathers(c0 + b + 2, b)

        return carry

    issue_gathers(0, 0)
    issue_gathers(1, 1)
    lax.fori_loop(0, NPAIR, pair, 0)
    scatter(0, 0).wait()
    scatter(1, 1).wait()

    plsc.subcore_barrier()
    pltpu.sync_copy(acc.at[pl.ds(sid * RPT, RPT)],
                    out_hbm.at[cid, pl.ds(sid * RPT, RPT)])


_sc_edge = pl.kernel(
    _sc_edge_body,
    out_type=jax.ShapeDtypeStruct((NC, NP, ROW), _f32),
    mesh=plsc.VectorSubcoreMesh(core_axis_name="c", subcore_axis_name="s"),
    scratch_types=[
        pltpu.VMEM((NCHUNK, CH), jnp.int32),
        pltpu.VMEM((NCHUNK, CH), jnp.int32),
        pltpu.VMEM((CH, ROW), _f32),
        pltpu.VMEM((CH, 16), _f32),
        pltpu.VMEM((CH, ROW), _f32),
        pltpu.VMEM((CH, 16), _f32),
        pltpu.SemaphoreType.DMA,
        pltpu.SemaphoreType.DMA,
        pltpu.SemaphoreType.DMA,
        pltpu.SemaphoreType.DMA,
        pltpu.VMEM_SHARED((NP, ROW), _f32),
    ],
    compiler_params=pltpu.CompilerParams(use_tc_tiling_on_sc=False),
)


def _finish_body(pu_ref, ps_ref, bu_ref, bs_ref, wl_ref, bl_ref, wc_ref,
                 bc_ref, out_ref, accs, accu, cnts, cntu):
    i = pl.program_id(0)

    @pl.when(i == 0)
    def _init():
        accs[...] = jnp.zeros_like(accs)
        accu[...] = jnp.zeros_like(accu)
        cnts[...] = jnp.zeros_like(cnts)
        cntu[...] = jnp.zeros_like(cntu)

    rows = lax.broadcasted_iota(jnp.int32, (H, C), 0)
    cols = lax.broadcasted_iota(jnp.int32, (H, C), 1)
    expand = (cols // D == rows).astype(_f32)

    def emb_of(p_ref):
        part = p_ref[0] + p_ref[1]
        num = part[:, :C]
        den = jnp.dot(part[:, C:C + H], expand, preferred_element_type=_f32,
                      precision=lax.Precision.HIGHEST)
        return jnp.maximum(num / (den + 1e-16), 0.0)

    emb_u = emb_of(pu_ref)
    emb_s = emb_of(ps_ref)

    giota = lax.broadcasted_iota(jnp.int32, (G, BN), 0)
    mask_u = (bu_ref[0] == giota).astype(_f32)
    mask_s = (bs_ref[0] == giota).astype(_f32)

    accu[...] += jnp.dot(mask_u, emb_u, preferred_element_type=_f32,
                         precision=lax.Precision.HIGHEST)
    accs[...] += jnp.dot(mask_s, emb_s, preferred_element_type=_f32,
                         precision=lax.Precision.HIGHEST)
    cntu[...] += jnp.broadcast_to(jnp.sum(mask_u, axis=1, keepdims=True),
                                  (G, C))
    cnts[...] += jnp.broadcast_to(jnp.sum(mask_s, axis=1, keepdims=True),
                                  (G, C))

    @pl.when(i == NB - 1)
    def _final():
        ps = accs[...] / jnp.maximum(cnts[...], 1.0)
        pu = accu[...] / jnp.maximum(cntu[...], 1.0)
        pooled = (ps + pu) * 0.5
        x = jnp.dot(pooled, wl_ref[...], preferred_element_type=_f32,
                    precision=lax.Precision.HIGHEST) + bl_ref[...]
        x = jnp.where(x > 0, x, 0.01 * x)
        out_ref[...] = jnp.dot(x, wc_ref[...], preferred_element_type=_f32,
                               precision=lax.Precision.HIGHEST) + bc_ref[...]


def _finish(p_user, p_source, batch_u3, batch_s3, w_lin, b_lin, w_cls, b_cls):
    return pl.pallas_call(
        _finish_body,
        grid=(NB,),
        in_specs=[
            pl.BlockSpec((NC, BN, ROW), lambda i: (0, i, 0)),
            pl.BlockSpec((NC, BN, ROW), lambda i: (0, i, 0)),
            pl.BlockSpec((1, 1, BN), lambda i: (i, 0, 0)),
            pl.BlockSpec((1, 1, BN), lambda i: (i, 0, 0)),
            pl.BlockSpec((C, C), lambda i: (0, 0)),
            pl.BlockSpec((1, C), lambda i: (0, 0)),
            pl.BlockSpec((C, 2), lambda i: (0, 0)),
            pl.BlockSpec((1, 2), lambda i: (0, 0)),
        ],
        out_specs=pl.BlockSpec((G, 2), lambda i: (0, 0)),
        out_shape=jax.ShapeDtypeStruct((G, 2), _f32),
        scratch_shapes=[pltpu.VMEM((G, C), _f32)] * 4,
    )(p_user, p_source, batch_u3, batch_s3, w_lin, b_lin, w_cls, b_cls)


def _att_mat(att):
    return (jnp.eye(H, dtype=_f32)[:, None, :] * att[:, :, None]).reshape(C, H)


def kernel(x_source, x_user, edge_index_su, edge_index_us, batch_source,
           batch_user, W_proj_source, b_proj_source, W_proj_user, b_proj_user,
           att_src_su, att_dst_su, att_src_us, att_dst_us, q, Wk, bk, W_lin,
           b_lin, W_cls, b_cls):
    z8 = jnp.zeros((C, H), _f32)
    wb_s = jnp.concatenate(
        [W_proj_source, W_proj_source @ _att_mat(att_src_su), z8,
         W_proj_source @ _att_mat(att_dst_us), z8], axis=1)
    wb_u = jnp.concatenate(
        [W_proj_user, W_proj_user @ _att_mat(att_src_us), z8,
         W_proj_user @ _att_mat(att_dst_su), z8], axis=1)
    zb = jnp.zeros((H,), _f32)
    bb_s = jnp.concatenate([b_proj_source, b_proj_source @ _att_mat(att_src_su),
                            zb, b_proj_source @ _att_mat(att_dst_us), zb])
    bb_u = jnp.concatenate([b_proj_user, b_proj_user @ _att_mat(att_src_us),
                            zb, b_proj_user @ _att_mat(att_dst_su), zb])

    s_big, s_small, u_big, u_small = _project(
        x_source, x_user, wb_s, wb_u, bb_s[None], bb_u[None])

    zeros_tile = jnp.zeros((RPT, ROW), _f32)
    p_user = _sc_edge(s_big, u_small,
                      edge_index_su[0].reshape(NW, NCHUNK, CH),
                      edge_index_su[1].reshape(NW, NCHUNK, CH), zeros_tile)
    p_source = _sc_edge(u_big, s_small,
                        edge_index_us[0].reshape(NW, NCHUNK, CH),
                        edge_index_us[1].reshape(NW, NCHUNK, CH), zeros_tile)

    return _finish(p_user, p_source,
                   batch_user.reshape(NB, 1, BN),
                   batch_source.reshape(NB, 1, BN),
                   W_lin, b_lin[None], W_cls, b_cls[None])


# speedup vs baseline: 30.4743x; 1.1687x over previous
import functools

import jax
import jax.numpy as jnp
from jax import lax
from jax.experimental import pallas as pl
from jax.experimental.pallas import tpu as pltpu
from jax.experimental.pallas import tpu_sc as plsc

N = 10000
C = 128
H = 8
D = 16
E = 320000
G = 64

ROW = 144
NB = 10
BN = N // NB
NC = 2
NS = 16
NW = NC * NS
PER_W = E // NW
CH = 40
NCHUNK = PER_W // CH
NPAIR = NCHUNK // 2
NP = 10240
RPT = NP // NS

_f32 = jnp.float32


def _proj_body(xs_ref, xu_ref, ws_ref, wu_ref, bs_ref, bu_ref,
               sbig_ref, ssml_ref, ubig_ref, usml_ref):
    rs = jnp.dot(xs_ref[...], ws_ref[...], preferred_element_type=_f32,
                 precision=lax.Precision.HIGHEST) + bs_ref[...]
    ru = jnp.dot(xu_ref[...], wu_ref[...], preferred_element_type=_f32,
                 precision=lax.Precision.HIGHEST) + bu_ref[...]
    sbig_ref[...] = rs[:, :ROW]
    ssml_ref[...] = rs[:, ROW:ROW + 16]
    ubig_ref[...] = ru[:, :ROW]
    usml_ref[...] = ru[:, ROW:ROW + 16]


def _project(x_s, x_u, wb_s, wb_u, bb_s, bb_u):
    return pl.pallas_call(
        _proj_body,
        grid=(NB,),
        in_specs=[
            pl.BlockSpec((BN, C), lambda i: (i, 0)),
            pl.BlockSpec((BN, C), lambda i: (i, 0)),
            pl.BlockSpec((C, ROW + 16), lambda i: (0, 0)),
            pl.BlockSpec((C, ROW + 16), lambda i: (0, 0)),
            pl.BlockSpec((1, ROW + 16), lambda i: (0, 0)),
            pl.BlockSpec((1, ROW + 16), lambda i: (0, 0)),
        ],
        out_specs=[
            pl.BlockSpec((BN, ROW), lambda i: (i, 0)),
            pl.BlockSpec((BN, 16), lambda i: (i, 0)),
            pl.BlockSpec((BN, ROW), lambda i: (i, 0)),
            pl.BlockSpec((BN, 16), lambda i: (i, 0)),
        ],
        out_shape=[
            jax.ShapeDtypeStruct((N, ROW), _f32),
            jax.ShapeDtypeStruct((N, 16), _f32),
            jax.ShapeDtypeStruct((N, ROW), _f32),
            jax.ShapeDtypeStruct((N, 16), _f32),
        ],
    )(x_s, x_u, wb_s, wb_u, bb_s, bb_u)


def _splat(v, h):
    idx = jnp.full((16, 1), h, jnp.int32)
    return lax.gather(
        v, idx,
        lax.GatherDimensionNumbers(offset_dims=(), collapsed_slice_dims=(0,),
                                   start_index_map=(0,)),
        (1,), mode=lax.GatherScatterMode.PROMISE_IN_BOUNDS)


NBUF = 4


def _sc_edge_body(big_hbm, small_hbm, sidx_hbm, didx_hbm, zeros_hbm, out_hbm,
                  sidxs, didx_all, bigs, smalls, semis, semgs, semcs, acc):
    cid = lax.axis_index("c")
    sid = lax.axis_index("s")
    wid = sid * NC + cid

    pltpu.sync_copy(zeros_hbm, acc.at[pl.ds(sid * RPT, RPT)])
    pltpu.sync_copy(didx_hbm.at[wid], didx_all)
    plsc.subcore_barrier()

    def sidx_copy(ci, b):
        return pltpu.make_async_copy(
            sidx_hbm.at[wid, pl.ds(ci * CH, CH)], sidxs[b], semis[b])

    def gathers(ci, b):
        return (pltpu.make_async_copy(big_hbm.at[sidxs[b]], bigs[b], semgs[b]),
                pltpu.make_async_copy(small_hbm.at[didx_all.at[ci]], smalls[b],
                                      semgs[b]))

    def scatter(ci, b):
        return pltpu.make_async_copy(bigs[b], acc.at[didx_all.at[ci]],
                                     semcs[b])

    def compute(b):
        big_v, small_v = bigs[b], smalls[b]

        @plsc.parallel_loop(0, CH, unroll=4)
        def _edges(e):
            t = big_v[e, pl.ds(C, 16)] + small_v[e, pl.ds(0, 16)]
            w = jnp.exp(jnp.maximum(t, 0.2 * t))
            for h in range(H):
                sp = _splat(w, h)
                big_v[e, pl.ds(D * h, D)] = big_v[e, pl.ds(D * h, D)] * sp
            big_v[e, pl.ds(C, 16)] = w

    def when(cond):
        if isinstance(cond, bool):
            return (lambda f: f() if cond else None)
        return pl.when(cond)

    def slot(ci, b):
        g1, g2 = gathers(ci, b)
        g1.wait()
        g2.wait()

        @when(ci + NBUF < NCHUNK)
        def _stage():
            sidx_copy(ci + NBUF, b).start()

        compute(b)
        scatter(ci, b).start(add=True)
        cg = ci + 2
        bg = (b + 2) % NBUF

        @when((ci >= 2) & (cg < NCHUNK) if not isinstance(ci, int)
              else (ci >= 2 and cg < NCHUNK))
        def _scatter_wait():
            scatter(cg, bg).wait()

        @when(cg < NCHUNK)
        def _issue():
            sidx_copy(cg, bg).wait()
            g1, g2 = gathers(cg, bg)
            g1.start()
            g2.start()

    for b in range(NBUF):
        sidx_copy(b, b).start()
    for b in range(2):
        sidx_copy(b, b).wait()
        g1, g2 = gathers(b, b)
        g1.start()
        g2.start()

    def ring_iter(j, carry):
        for b in range(NBUF):
            slot(NBUF * j + b, b)
        return carry

    lax.fori_loop(0, NCHUNK // NBUF, ring_iter, 0)
    for k in range(NCHUNK // NBUF * NBUF, NCHUNK):
        slot(k, k % NBUF)
    for k in range(NCHUNK - 2, NCHUNK):
        scatter(k, k % NBUF).wait()

    plsc.subcore_barrier()
    pltpu.sync_copy(acc.at[pl.ds(sid * RPT, RPT)],
                    out_hbm.at[cid, pl.ds(sid * RPT, RPT)])


_sc_edge = pl.kernel(
    _sc_edge_body,
    out_type=jax.ShapeDtypeStruct((NC, NP, ROW), _f32),
    mesh=plsc.VectorSubcoreMesh(core_axis_name="c", subcore_axis_name="s"),
    scratch_types=[
        [pltpu.VMEM((CH,), jnp.int32)] * NBUF,
        pltpu.VMEM((NCHUNK, CH), jnp.int32),
        [pltpu.VMEM((CH, ROW), _f32)] * NBUF,
        [pltpu.VMEM((CH, 16), _f32)] * NBUF,
        [pltpu.SemaphoreType.DMA] * NBUF,
        [pltpu.SemaphoreType.DMA] * NBUF,
        [pltpu.SemaphoreType.DMA] * NBUF,
        pltpu.VMEM_SHARED((NP, ROW), _f32),
    ],
    compiler_params=pltpu.CompilerParams(use_tc_tiling_on_sc=False),
)


def _finish_body(pu_ref, ps_ref, bu_ref, bs_ref, wl_ref, bl_ref, wc_ref,
                 bc_ref, out_ref, accs, accu, cnts, cntu):
    i = pl.program_id(0)

    @pl.when(i == 0)
    def _init():
        accs[...] = jnp.zeros_like(accs)
        accu[...] = jnp.zeros_like(accu)
        cnts[...] = jnp.zeros_like(cnts)
        cntu[...] = jnp.zeros_like(cntu)

    rows = lax.broadcasted_iota(jnp.int32, (H, C), 0)
    cols = lax.broadcasted_iota(jnp.int32, (H, C), 1)
    expand = (cols // D == rows).astype(_f32)

    def emb_of(p_ref):
        part = p_ref[0] + p_ref[1]
        num = part[:, :C]
        den = jnp.dot(part[:, C:C + H], expand, preferred_element_type=_f32,
                      precision=lax.Precision.HIGHEST)
        return jnp.maximum(num / (den + 1e-16), 0.0)

    emb_u = emb_of(pu_ref)
    emb_s = emb_of(ps_ref)

    giota = lax.broadcasted_iota(jnp.int32, (G, BN), 0)
    mask_u = (bu_ref[0] == giota).astype(_f32)
    mask_s = (bs_ref[0] == giota).astype(_f32)

    accu[...] += jnp.dot(mask_u, emb_u, preferred_element_type=_f32,
                         precision=lax.Precision.HIGHEST)
    accs[...] += jnp.dot(mask_s, emb_s, preferred_element_type=_f32,
                         precision=lax.Precision.HIGHEST)
    cntu[...] += jnp.broadcast_to(jnp.sum(mask_u, axis=1, keepdims=True),
                                  (G, C))
    cnts[...] += jnp.broadcast_to(jnp.sum(mask_s, axis=1, keepdims=True),
                                  (G, C))

    @pl.when(i == NB - 1)
    def _final():
        ps = accs[...] / jnp.maximum(cnts[...], 1.0)
        pu = accu[...] / jnp.maximum(cntu[...], 1.0)
        pooled = (ps + pu) * 0.5
        x = jnp.dot(pooled, wl_ref[...], preferred_element_type=_f32,
                    precision=lax.Precision.HIGHEST) + bl_ref[...]
        x = jnp.where(x > 0, x, 0.01 * x)
        out_ref[...] = jnp.dot(x, wc_ref[...], preferred_element_type=_f32,
                               precision=lax.Precision.HIGHEST) + bc_ref[...]


def _finish(p_user, p_source, batch_u3, batch_s3, w_lin, b_lin, w_cls, b_cls):
    return pl.pallas_call(
        _finish_body,
        grid=(NB,),
        in_specs=[
            pl.BlockSpec((NC, BN, ROW), lambda i: (0, i, 0)),
            pl.BlockSpec((NC, BN, ROW), lambda i: (0, i, 0)),
            pl.BlockSpec((1, 1, BN), lambda i: (i, 0, 0)),
            pl.BlockSpec((1, 1, BN), lambda i: (i, 0, 0)),
            pl.BlockSpec((C, C), lambda i: (0, 0)),
            pl.BlockSpec((1, C), lambda i: (0, 0)),
            pl.BlockSpec((C, 2), lambda i: (0, 0)),
            pl.BlockSpec((1, 2), lambda i: (0, 0)),
        ],
        out_specs=pl.BlockSpec((G, 2), lambda i: (0, 0)),
        out_shape=jax.ShapeDtypeStruct((G, 2), _f32),
        scratch_shapes=[pltpu.VMEM((G, C), _f32)] * 4,
    )(p_user, p_source, batch_u3, batch_s3, w_lin, b_lin, w_cls, b_cls)


def _att_mat(att):
    return (jnp.eye(H, dtype=_f32)[:, None, :] * att[:, :, None]).reshape(C, H)


def kernel(x_source, x_user, edge_index_su, edge_index_us, batch_source,
           batch_user, W_proj_source, b_proj_source, W_proj_user, b_proj_user,
           att_src_su, att_dst_su, att_src_us, att_dst_us, q, Wk, bk, W_lin,
           b_lin, W_cls, b_cls):
    z8 = jnp.zeros((C, H), _f32)
    wb_s = jnp.concatenate(
        [W_proj_source, W_proj_source @ _att_mat(att_src_su), z8,
         W_proj_source @ _att_mat(att_dst_us), z8], axis=1)
    wb_u = jnp.concatenate(
        [W_proj_user, W_proj_user @ _att_mat(att_src_us), z8,
         W_proj_user @ _att_mat(att_dst_su), z8], axis=1)
    zb = jnp.zeros((H,), _f32)
    bb_s = jnp.concatenate([b_proj_source, b_proj_source @ _att_mat(att_src_su),
                            zb, b_proj_source @ _att_mat(att_dst_us), zb])
    bb_u = jnp.concatenate([b_proj_user, b_proj_user @ _att_mat(att_src_us),
                            zb, b_proj_user @ _att_mat(att_dst_su), zb])

    s_big, s_small, u_big, u_small = _project(
        x_source, x_user, wb_s, wb_u, bb_s[None], bb_u[None])

    zeros_tile = jnp.zeros((RPT, ROW), _f32)
    p_user = _sc_edge(s_big, u_small,
                      edge_index_su[0].reshape(NW, PER_W),
                      edge_index_su[1].reshape(NW, NCHUNK, CH), zeros_tile)
    p_source = _sc_edge(u_big, s_small,
                        edge_index_us[0].reshape(NW, PER_W),
                        edge_index_us[1].reshape(NW, NCHUNK, CH), zeros_tile)

    return _finish(p_user, p_source,
                   batch_user.reshape(NB, 1, BN),
                   batch_source.reshape(NB, 1, BN),
                   W_lin, b_lin[None], W_cls, b_cls[None])
